# baseline (speedup 1.0000x reference)
import functools

import jax
import jax.numpy as jnp
from jax import lax
from jax.experimental import pallas as pl
from jax.experimental.pallas import tpu as pltpu
from jax.experimental.pallas import tpu_sc as plsc

N = 10000
E = 320000
D = 128
H = 128
C = 10
G = 64

NC = 2
NS = 16
L = 16
QW = 32

K = 80
EPT = E // NS
CHUNKS = EPT // K
ROWS_E = E // K
NPT = N // NS
RPW = ROWS_E // (NC * NS)
MK = 125
MCHUNKS = EPT // MK

_MESH = plsc.VectorSubcoreMesh(core_axis_name="c", subcore_axis_name="s")
_SC_PARAMS = pltpu.CompilerParams(needs_layout_passes=False,
                                  use_tc_tiling_on_sc=False,
                                  disable_bounds_checks=True,
                                  disable_semaphore_checks=True)


@functools.partial(
    pl.kernel,
    out_type=jax.ShapeDtypeStruct((NC, NS, NPT, L), jnp.float32),
    mesh=_MESH,
    compiler_params=_SC_PARAMS,
    scratch_types=[
        pltpu.VMEM((CHUNKS, K), jnp.int32),
        pltpu.VMEM((CHUNKS * K,), jnp.float32),
        pltpu.VMEM((K, L), jnp.float32),
        pltpu.VMEM((NPT, L), jnp.float32),
        pltpu.VMEM_SHARED((N, L), jnp.float32),
    ],
)
def _deg_kernel(src_hbm, dst_hbm, wf_hbm, deg_hbm, idx_v, w_v, pad_v, zero_v,
                acc):
    c = lax.axis_index("c")
    s = lax.axis_index("s")

    @pl.when(c == 0)
    def _():
        pltpu.sync_copy(src_hbm.at[s], idx_v)

    @pl.when(c == 1)
    def _():
        pltpu.sync_copy(dst_hbm.at[s], idx_v)

    pltpu.sync_copy(wf_hbm.at[s], w_v)

    z16 = jnp.zeros((L,), jnp.float32)

    def zrow(i, _):
        zero_v[i, :] = z16
        return 0

    lax.fori_loop(0, NPT, zrow, 0)
    pltpu.sync_copy(zero_v, acc.at[pl.ds(s * NPT, NPT)])
    plsc.subcore_barrier()

    def chunk(g, _):
        gK = g * K

        def row(r, _):
            pad_v[r, :] = plsc.load_gather(
                w_v, [jnp.full((L,), gK + r, jnp.int32)])
            return 0

        lax.fori_loop(0, K, row, 0, unroll=8)
        pltpu.sync_copy(pad_v, acc.at[idx_v.at[g]], add=True)
        return 0

    lax.fori_loop(0, CHUNKS, chunk, 0)

    plsc.subcore_barrier()
    pltpu.sync_copy(acc.at[pl.ds(s * NPT, NPT)], deg_hbm.at[c, s])


def _rsqrt_body(ds_ref, dd_ref, rsd_ref, rdd_ref):
    rsd_ref[...] = lax.rsqrt(jnp.maximum(ds_ref[0][:, :1], 1e-12))
    rdd_ref[...] = lax.rsqrt(jnp.maximum(dd_ref[0][:, :1], 1e-12))


def _rsqrt_call(deg_pad):
    blk = 1000
    return pl.pallas_call(
        _rsqrt_body,
        grid=(N // blk,),
        in_specs=[
            pl.BlockSpec((1, blk, L), lambda i: (0, i, 0)),
            pl.BlockSpec((1, blk, L), lambda i: (1, i, 0)),
        ],
        out_specs=[
            pl.BlockSpec((blk, 1), lambda i: (i, 0)),
            pl.BlockSpec((blk, 1), lambda i: (i, 0)),
        ],
        out_shape=[
            jax.ShapeDtypeStruct((N, 1), jnp.float32),
            jax.ShapeDtypeStruct((N, 1), jnp.float32),
        ],
    )(deg_pad, deg_pad)


@functools.partial(
    pl.kernel,
    out_type=jax.ShapeDtypeStruct((NC * NS, RPW, K), jnp.float32),
    mesh=_MESH,
    compiler_params=_SC_PARAMS,
    scratch_types=[
        pltpu.VMEM((N,), jnp.float32),
        pltpu.VMEM((N,), jnp.float32),
        pltpu.VMEM((RPW, K), jnp.int32),
        pltpu.VMEM((RPW, K), jnp.int32),
        pltpu.VMEM((RPW, K), jnp.float32),
        pltpu.VMEM((RPW, K), jnp.float32),
    ],
)
def _norm_kernel(src_hbm, dst_hbm, w_hbm, rsd_hbm, rdd_hbm, norm_hbm,
                 rsd_v, rdd_v, src_v, dst_v, w_v, norm_v):
    c = lax.axis_index("c")
    s = lax.axis_index("s")
    wid = s * NC + c

    pltpu.sync_copy(rsd_hbm, rsd_v)
    pltpu.sync_copy(rdd_hbm, rdd_v)
    pltpu.sync_copy(src_hbm.at[wid], src_v)
    pltpu.sync_copy(dst_hbm.at[wid], dst_v)
    pltpu.sync_copy(w_hbm.at[wid], w_v)

    def row(g, _):
        for j in range(K // L):
            sl = pl.ds(j * L, L)
            a = plsc.load_gather(rsd_v, [src_v[g, sl]])
            b = plsc.load_gather(rdd_v, [dst_v[g, sl]])
            norm_v[g, sl] = w_v[g, sl] * a * b
        return 0

    lax.fori_loop(0, RPW, row, 0, unroll=4)
    pltpu.sync_copy(norm_v, norm_hbm.at[wid])


@functools.partial(
    pl.kernel,
    out_type=jax.ShapeDtypeStruct((2, NC, NS, NPT, QW), jnp.float32),
    mesh=_MESH,
    compiler_params=_SC_PARAMS,
    scratch_types=[
        pltpu.VMEM((MCHUNKS, MK), jnp.int32),
        pltpu.VMEM((MCHUNKS, MK), jnp.int32),
        pltpu.VMEM((MCHUNKS * MK,), jnp.float32),
        pltpu.VMEM((MK, QW), jnp.float32),
        pltpu.VMEM((MK, QW), jnp.float32),
        pltpu.VMEM((MK, QW), jnp.float32),
        pltpu.VMEM((MK, QW), jnp.float32),
        pltpu.VMEM((NPT, QW), jnp.float32),
        pltpu.SemaphoreType.DMA,
        pltpu.SemaphoreType.DMA,
        pltpu.SemaphoreType.DMA,
        pltpu.SemaphoreType.DMA,
        pltpu.SemaphoreType.DMA,
        pltpu.SemaphoreType.DMA,
        pltpu.SemaphoreType.DMA,
        pltpu.SemaphoreType.DMA,
        pltpu.VMEM_SHARED((N, QW), jnp.float32),
    ],
)
def _msg_kernel(h0_hbm, h1_hbm, h2_hbm, h3_hbm, src_hbm, dst_hbm,
                normf_hbm, out_hbm,
                src_v, dst_v, norm_v, rows0, rows1, rows2, rows3, zero_v,
                gs0, gs1, gs2, gs3, ss0, ss1, ss2, ss3, acc):
    c = lax.axis_index("c")
    s = lax.axis_index("s")

    pltpu.sync_copy(src_hbm.at[s], src_v)
    pltpu.sync_copy(dst_hbm.at[s], dst_v)
    pltpu.sync_copy(normf_hbm.at[s], norm_v)

    z16 = jnp.zeros((L,), jnp.float32)

    def zrow(i, _):
        for j in range(QW // L):
            zero_v[i, pl.ds(j * L, L)] = z16
        return 0

    lax.fori_loop(0, NPT, zrow, 0, unroll=8)

    bufs = ((rows0, gs0, ss0), (rows1, gs1, ss1), (rows2, gs2, ss2),
            (rows3, gs3, ss3))

    def body(h2_ref):
        for b in range(2):
            rv, gsm, _ = bufs[b]
            pltpu.async_copy(h2_ref.at[src_v.at[b]], rv, gsm)

        def quad(q, _):
            g0 = q * 4
            for b in range(4):
                rv, gsm, ssm = bufs[b]
                g = g0 + b
                pltpu.make_async_copy(h2_ref.at[src_v.at[g]], rv, gsm).wait()
                gK = g * MK

                def row(r, _):
                    nrm = plsc.load_gather(
                        norm_v, [jnp.full((L,), gK + r, jnp.int32)])
                    for j in range(QW // L):
                        sl = pl.ds(j * L, L)
                        rv[r, sl] = rv[r, sl] * nrm
                    return 0

                lax.fori_loop(0, MK, row, 0, unroll=8)
                pltpu.async_copy(rv, acc.at[dst_v.at[g]], ssm, add=True)

                rv2, gsm2, ssm2 = bufs[(b + 2) % 4]

                @pl.when(g >= 2)
                def _():
                    pltpu.make_async_copy(
                        rv2, acc.at[dst_v.at[g - 2]], ssm2).wait()

                @pl.when(g + 2 < MCHUNKS)
                def _():
                    pltpu.async_copy(h2_ref.at[src_v.at[g + 2]], rv2, gsm2)
            return 0

        lax.fori_loop(0, MCHUNKS // 4, quad, 0)

        for g in (MCHUNKS - 2, MCHUNKS - 1):
            rv, _, ssm = bufs[g % 4]
            pltpu.make_async_copy(rv, acc.at[dst_v.at[g]], ssm).wait()

    for p, (ha, hb) in enumerate(((h0_hbm, h1_hbm), (h2_hbm, h3_hbm))):
        pltpu.sync_copy(zero_v, acc.at[pl.ds(s * NPT, NPT)])
        plsc.subcore_barrier()

        @pl.when(c == 0)
        def _():
            body(ha)

        @pl.when(c == 1)
        def _():
            body(hb)

        plsc.subcore_barrier()
        pltpu.sync_copy(acc.at[pl.ds(s * NPT, NPT)], out_hbm.at[p, c, s])


BLK = 1000


def _elu(v):
    return jnp.where(v > 0, v, jnp.exp(jnp.minimum(v, 0.0)) - 1.0)


def _pre(q_refs, bias_ref, alpha_ref):
    xin = jnp.concatenate([q[0] for q in q_refs], axis=1)
    v = xin + bias_ref[...]
    return jnp.where(alpha_ref[...] > 0.0, _elu(v), v)


def _mid_body(a_ref, b_ref, c_ref, d_ref, bias_ref, w_ref, alpha_ref,
              h_ref):
    xin = _pre((a_ref, b_ref, c_ref, d_ref), bias_ref, alpha_ref)
    h = jnp.dot(xin, w_ref[...], preferred_element_type=jnp.float32)
    for q in range(4):
        h_ref[q] = h[:, q * QW:(q + 1) * QW]


def _mid_call(accs, bias, Wn, alpha):
    return pl.pallas_call(
        _mid_body,
        grid=(N // BLK,),
        in_specs=[
            pl.BlockSpec((1, BLK, QW), lambda i: (0, i, 0)),
            pl.BlockSpec((1, BLK, QW), lambda i: (1, i, 0)),
            pl.BlockSpec((1, BLK, QW), lambda i: (2, i, 0)),
            pl.BlockSpec((1, BLK, QW), lambda i: (3, i, 0)),
            pl.BlockSpec((1, H), lambda i: (0, 0)),
            pl.BlockSpec((H, H), lambda i: (0, 0)),
            pl.BlockSpec((1, 1), lambda i: (0, 0)),
        ],
        out_specs=pl.BlockSpec((4, BLK, QW), lambda i: (0, i, 0)),
        out_shape=jax.ShapeDtypeStruct((4, N, QW), jnp.float32),
    )(accs, accs, accs, accs, bias, Wn, alpha)


def _final_body(a_ref, b_ref, c_ref, d_ref, bias_ref, batch_ref,
                wl_ref, bl_ref, one_ref, out_ref, seg_ref, cnt_ref):
    i = pl.program_id(0)

    @pl.when(i == 0)
    def _():
        seg_ref[...] = jnp.zeros_like(seg_ref)
        cnt_ref[...] = jnp.zeros_like(cnt_ref)

    x4 = _pre((a_ref, b_ref, c_ref, d_ref), bias_ref, one_ref)
    brow = batch_ref[0]
    onehot = (lax.broadcasted_iota(jnp.int32, (G, BLK), 0) == brow
              ).astype(jnp.float32)
    seg_ref[...] += jnp.dot(onehot, x4, preferred_element_type=jnp.float32)
    cnt_ref[...] += jnp.sum(onehot, axis=1, keepdims=True)

    @pl.when(i == pl.num_programs(0) - 1)
    def _():
        pooled = seg_ref[...] / jnp.maximum(cnt_ref[...], 1.0)
        out_ref[...] = (
            jnp.dot(pooled, wl_ref[...], preferred_element_type=jnp.float32)
            + bl_ref[...])


def _final_call(accs, bias, batch3, Wl, bl):
    one = jnp.ones((1, 1), jnp.float32)
    return pl.pallas_call(
        _final_body,
        grid=(N // BLK,),
        in_specs=[
            pl.BlockSpec((1, BLK, QW), lambda i: (0, i, 0)),
            pl.BlockSpec((1, BLK, QW), lambda i: (1, i, 0)),
            pl.BlockSpec((1, BLK, QW), lambda i: (2, i, 0)),
            pl.BlockSpec((1, BLK, QW), lambda i: (3, i, 0)),
            pl.BlockSpec((1, H), lambda i: (0, 0)),
            pl.BlockSpec((1, 1, BLK), lambda i: (i, 0, 0)),
            pl.BlockSpec((H, C), lambda i: (0, 0)),
            pl.BlockSpec((1, C), lambda i: (0, 0)),
            pl.BlockSpec((1, 1), lambda i: (0, 0)),
        ],
        out_specs=pl.BlockSpec((G, C), lambda i: (0, 0)),
        out_shape=jax.ShapeDtypeStruct((G, C), jnp.float32),
        scratch_shapes=[
            pltpu.VMEM((G, H), jnp.float32),
            pltpu.VMEM((G, 1), jnp.float32),
        ],
        compiler_params=pltpu.CompilerParams(
            dimension_semantics=("arbitrary",)),
    )(accs, accs, accs, accs, bias, batch3, Wl, bl, one)


@jax.jit
def kernel(x, edge_index, edge_type, edge_attr, batch,
           W1, b1, W2, b2, W3, b3, W4, b4, Wl, bl):
    src = edge_index[0].reshape(NS, MCHUNKS, MK)
    dst = edge_index[1].reshape(NS, MCHUNKS, MK)
    srcd = edge_index[0].reshape(NS, CHUNKS, K)
    dstd = edge_index[1].reshape(NS, CHUNKS, K)
    wf = edge_attr[:, 0].reshape(NS, CHUNKS * K)
    srcn = edge_index[0].reshape(NC * NS, RPW, K)
    dstn = edge_index[1].reshape(NC * NS, RPW, K)
    wn = edge_attr[:, 0].reshape(NC * NS, RPW, K)

    deg_pad = _deg_kernel(srcd, dstd, wf).reshape(NC, N, L)
    rsd, rdd = _rsqrt_call(deg_pad)
    norm = _norm_kernel(srcn, dstn, wn, rsd.reshape(N),
                        rdd.reshape(N)).reshape(NS, CHUNKS * K)

    accs0 = jnp.stack([x[:, q * QW:(q + 1) * QW] for q in range(4)])
    Ws = jnp.stack([W1, W2, W3, W4])
    bs = jnp.stack([jnp.zeros_like(b1), b1, b2, b3]).reshape(4, 1, H)
    alphas = jnp.array([0.0, 1.0, 1.0, 1.0], jnp.float32).reshape(4, 1, 1)

    def _layer_step(accs, wba):
        Wn, bprev, alpha = wba
        hq = _mid_call(accs, bprev, Wn, alpha)
        out = _msg_kernel(hq[0], hq[1], hq[2], hq[3], src, dst, norm)
        return out.reshape(4, N, QW), None

    accs, _ = lax.scan(_layer_step, accs0, (Ws, bs, alphas))

    batch3 = batch.reshape(N // BLK, 1, BLK)
    return _final_call(accs, b4.reshape(1, H), batch3, Wl,
                       bl.reshape(1, C))


# speedup vs baseline: 1.0203x; 1.0203x over previous
import functools

import jax
import jax.numpy as jnp
from jax import lax
from jax.experimental import pallas as pl
from jax.experimental.pallas import tpu as pltpu
from jax.experimental.pallas import tpu_sc as plsc

N = 10000
E = 320000
D = 128
H = 128
C = 10
G = 64

NC = 2
NS = 16
L = 16
QW = 32

K = 80
EPT = E // NS
CHUNKS = EPT // K
ROWS_E = E // K
NPT = N // NS
RPW = ROWS_E // (NC * NS)
MK = 125
MCHUNKS = EPT // MK

_MESH = plsc.VectorSubcoreMesh(core_axis_name="c", subcore_axis_name="s")
_SC_PARAMS = pltpu.CompilerParams(needs_layout_passes=False,
                                  use_tc_tiling_on_sc=False,
                                  disable_bounds_checks=True,
                                  disable_semaphore_checks=True)


@functools.partial(
    pl.kernel,
    out_type=jax.ShapeDtypeStruct((NC, NS, NPT, L), jnp.float32),
    mesh=_MESH,
    compiler_params=_SC_PARAMS,
    scratch_types=[
        pltpu.VMEM((CHUNKS, K), jnp.int32),
        pltpu.VMEM((CHUNKS * K,), jnp.float32),
        pltpu.VMEM((K, L), jnp.float32),
        pltpu.VMEM((K, L), jnp.float32),
        pltpu.VMEM((NPT, L), jnp.float32),
        pltpu.SemaphoreType.DMA,
        pltpu.SemaphoreType.DMA,
        pltpu.VMEM_SHARED((N, L), jnp.float32),
    ],
)
def _deg_kernel(src_hbm, dst_hbm, wf_hbm, deg_hbm, idx_v, w_v, pad0, pad1,
                zero_v, ps0, ps1, acc):
    c = lax.axis_index("c")
    s = lax.axis_index("s")

    @pl.when(c == 0)
    def _():
        pltpu.sync_copy(src_hbm.at[s], idx_v)

    @pl.when(c == 1)
    def _():
        pltpu.sync_copy(dst_hbm.at[s], idx_v)

    pltpu.sync_copy(wf_hbm.at[s], w_v)

    z16 = jnp.zeros((L,), jnp.float32)

    def zrow(i, _):
        zero_v[i, :] = z16
        return 0

    lax.fori_loop(0, NPT, zrow, 0)
    pltpu.sync_copy(zero_v, acc.at[pl.ds(s * NPT, NPT)])
    plsc.subcore_barrier()

    pads = ((pad0, ps0), (pad1, ps1))

    def pair(g2, _):
        for b in range(2):
            pv, sm = pads[b]
            g = g2 * 2 + b
            gK = g * K

            @pl.when(g >= 2)
            def _():
                pltpu.make_async_copy(pv, acc.at[idx_v.at[g - 2]], sm).wait()

            def row(r, _):
                pv[r, :] = plsc.load_gather(
                    w_v, [jnp.full((L,), gK + r, jnp.int32)])
                return 0

            lax.fori_loop(0, K, row, 0, unroll=8)
            pltpu.async_copy(pv, acc.at[idx_v.at[g]], sm, add=True)
        return 0

    lax.fori_loop(0, CHUNKS // 2, pair, 0)
    for g in (CHUNKS - 2, CHUNKS - 1):
        pv, sm = pads[g % 2]
        pltpu.make_async_copy(pv, acc.at[idx_v.at[g]], sm).wait()

    plsc.subcore_barrier()
    pltpu.sync_copy(acc.at[pl.ds(s * NPT, NPT)], deg_hbm.at[c, s])


def _rsqrt_body(ds_ref, dd_ref, rsd_ref, rdd_ref):
    rsd_ref[...] = lax.rsqrt(jnp.maximum(ds_ref[0][:, :1], 1e-12))
    rdd_ref[...] = lax.rsqrt(jnp.maximum(dd_ref[0][:, :1], 1e-12))


def _rsqrt_call(deg_pad):
    blk = 1000
    return pl.pallas_call(
        _rsqrt_body,
        grid=(N // blk,),
        in_specs=[
            pl.BlockSpec((1, blk, L), lambda i: (0, i, 0)),
            pl.BlockSpec((1, blk, L), lambda i: (1, i, 0)),
        ],
        out_specs=[
            pl.BlockSpec((blk, 1), lambda i: (i, 0)),
            pl.BlockSpec((blk, 1), lambda i: (i, 0)),
        ],
        out_shape=[
            jax.ShapeDtypeStruct((N, 1), jnp.float32),
            jax.ShapeDtypeStruct((N, 1), jnp.float32),
        ],
    )(deg_pad, deg_pad)


def _rsqrt16(x):
    i = plsc.bitcast(x, jnp.int32)
    i = jnp.int32(0x5F3759DF) - lax.shift_right_logical(i, 1)
    y = plsc.bitcast(i, jnp.float32)
    xh = x * 0.5
    for _ in range(3):
        y = y * (1.5 - xh * y * y)
    return y


@functools.partial(
    pl.kernel,
    out_type=jax.ShapeDtypeStruct((NC * NS, RPW, K), jnp.float32),
    mesh=_MESH,
    compiler_params=_SC_PARAMS,
    scratch_types=[
        pltpu.VMEM((N,), jnp.float32),
        pltpu.VMEM((N,), jnp.float32),
        pltpu.VMEM((RPW, K), jnp.int32),
        pltpu.VMEM((RPW, K), jnp.int32),
        pltpu.VMEM((RPW, K), jnp.float32),
        pltpu.VMEM((RPW, K), jnp.float32),
    ],
)
def _norm_kernel(src_hbm, dst_hbm, w_hbm, ds_hbm, dd_hbm, norm_hbm,
                 rsd_v, rdd_v, src_v, dst_v, w_v, norm_v):
    c = lax.axis_index("c")
    s = lax.axis_index("s")
    wid = s * NC + c

    pltpu.sync_copy(ds_hbm, rsd_v)
    pltpu.sync_copy(dd_hbm, rdd_v)
    pltpu.sync_copy(src_hbm.at[wid], src_v)
    pltpu.sync_copy(dst_hbm.at[wid], dst_v)
    pltpu.sync_copy(w_hbm.at[wid], w_v)

    def rrow(i, _):
        sl = pl.ds(i * L, L)
        rsd_v[sl] = _rsqrt16(jnp.maximum(rsd_v[sl], 1e-12))
        rdd_v[sl] = _rsqrt16(jnp.maximum(rdd_v[sl], 1e-12))
        return 0

    lax.fori_loop(0, N // L, rrow, 0, unroll=4)

    def row(g, _):
        for j in range(K // L):
            sl = pl.ds(j * L, L)
            a = plsc.load_gather(rsd_v, [src_v[g, sl]])
            b = plsc.load_gather(rdd_v, [dst_v[g, sl]])
            norm_v[g, sl] = w_v[g, sl] * a * b
        return 0

    lax.fori_loop(0, RPW, row, 0, unroll=4)
    pltpu.sync_copy(norm_v, norm_hbm.at[wid])


@functools.partial(
    pl.kernel,
    out_type=jax.ShapeDtypeStruct((2, NC, NS, NPT, QW), jnp.float32),
    mesh=_MESH,
    compiler_params=_SC_PARAMS,
    scratch_types=[
        pltpu.VMEM((MCHUNKS, MK), jnp.int32),
        pltpu.VMEM((MCHUNKS, MK), jnp.int32),
        pltpu.VMEM((MCHUNKS * MK,), jnp.float32),
        pltpu.VMEM((MK, QW), jnp.float32),
        pltpu.VMEM((MK, QW), jnp.float32),
        pltpu.VMEM((MK, QW), jnp.float32),
        pltpu.VMEM((MK, QW), jnp.float32),
        pltpu.VMEM((NPT, QW), jnp.float32),
        pltpu.SemaphoreType.DMA,
        pltpu.SemaphoreType.DMA,
        pltpu.SemaphoreType.DMA,
        pltpu.SemaphoreType.DMA,
        pltpu.SemaphoreType.DMA,
        pltpu.SemaphoreType.DMA,
        pltpu.SemaphoreType.DMA,
        pltpu.SemaphoreType.DMA,
        pltpu.VMEM_SHARED((N, QW), jnp.float32),
    ],
)
def _msg_kernel(h0_hbm, h1_hbm, h2_hbm, h3_hbm, src_hbm, dst_hbm,
                normf_hbm, out_hbm,
                src_v, dst_v, norm_v, rows0, rows1, rows2, rows3, zero_v,
                gs0, gs1, gs2, gs3, ss0, ss1, ss2, ss3, acc):
    c = lax.axis_index("c")
    s = lax.axis_index("s")

    pltpu.sync_copy(src_hbm.at[s], src_v)
    pltpu.sync_copy(dst_hbm.at[s], dst_v)
    pltpu.sync_copy(normf_hbm.at[s], norm_v)

    z16 = jnp.zeros((L,), jnp.float32)

    def zrow(i, _):
        for j in range(QW // L):
            zero_v[i, pl.ds(j * L, L)] = z16
        return 0

    lax.fori_loop(0, NPT, zrow, 0, unroll=8)

    bufs = ((rows0, gs0, ss0), (rows1, gs1, ss1), (rows2, gs2, ss2),
            (rows3, gs3, ss3))

    def body(h2_ref):
        for b in range(2):
            rv, gsm, _ = bufs[b]
            pltpu.async_copy(h2_ref.at[src_v.at[b]], rv, gsm)

        def quad(q, _):
            g0 = q * 4
            for b in range(4):
                rv, gsm, ssm = bufs[b]
                g = g0 + b
                pltpu.make_async_copy(h2_ref.at[src_v.at[g]], rv, gsm).wait()
                gK = g * MK

                def row(r, _):
                    nrm = plsc.load_gather(
                        norm_v, [jnp.full((L,), gK + r, jnp.int32)])
                    for j in range(QW // L):
                        sl = pl.ds(j * L, L)
                        rv[r, sl] = rv[r, sl] * nrm
                    return 0

                lax.fori_loop(0, MK, row, 0, unroll=8)
                pltpu.async_copy(rv, acc.at[dst_v.at[g]], ssm, add=True)

                rv2, gsm2, ssm2 = bufs[(b + 2) % 4]

                @pl.when(g >= 2)
                def _():
                    pltpu.make_async_copy(
                        rv2, acc.at[dst_v.at[g - 2]], ssm2).wait()

                @pl.when(g + 2 < MCHUNKS)
                def _():
                    pltpu.async_copy(h2_ref.at[src_v.at[g + 2]], rv2, gsm2)
            return 0

        lax.fori_loop(0, MCHUNKS // 4, quad, 0)

        for g in (MCHUNKS - 2, MCHUNKS - 1):
            rv, _, ssm = bufs[g % 4]
            pltpu.make_async_copy(rv, acc.at[dst_v.at[g]], ssm).wait()

    for p, (ha, hb) in enumerate(((h0_hbm, h1_hbm), (h2_hbm, h3_hbm))):
        pltpu.sync_copy(zero_v, acc.at[pl.ds(s * NPT, NPT)])
        plsc.subcore_barrier()

        @pl.when(c == 0)
        def _():
            body(ha)

        @pl.when(c == 1)
        def _():
            body(hb)

        plsc.subcore_barrier()
        pltpu.sync_copy(acc.at[pl.ds(s * NPT, NPT)], out_hbm.at[p, c, s])


BLK = 1000


def _elu(v):
    return jnp.where(v > 0, v, jnp.exp(jnp.minimum(v, 0.0)) - 1.0)


def _pre(q_refs, bias_ref, alpha_ref):
    xin = jnp.concatenate([q[0] for q in q_refs], axis=1)
    v = xin + bias_ref[...]
    return jnp.where(alpha_ref[...] > 0.0, _elu(v), v)


def _mid_body(a_ref, b_ref, c_ref, d_ref, bias_ref, w_ref, alpha_ref,
              h_ref):
    xin = _pre((a_ref, b_ref, c_ref, d_ref), bias_ref, alpha_ref)
    h = jnp.dot(xin, w_ref[...], preferred_element_type=jnp.float32)
    for q in range(4):
        h_ref[q] = h[:, q * QW:(q + 1) * QW]


def _mid_call(accs, bias, Wn, alpha):
    return pl.pallas_call(
        _mid_body,
        grid=(N // BLK,),
        in_specs=[
            pl.BlockSpec((1, BLK, QW), lambda i: (0, i, 0)),
            pl.BlockSpec((1, BLK, QW), lambda i: (1, i, 0)),
            pl.BlockSpec((1, BLK, QW), lambda i: (2, i, 0)),
            pl.BlockSpec((1, BLK, QW), lambda i: (3, i, 0)),
            pl.BlockSpec((1, H), lambda i: (0, 0)),
            pl.BlockSpec((H, H), lambda i: (0, 0)),
            pl.BlockSpec((1, 1), lambda i: (0, 0)),
        ],
        out_specs=pl.BlockSpec((4, BLK, QW), lambda i: (0, i, 0)),
        out_shape=jax.ShapeDtypeStruct((4, N, QW), jnp.float32),
    )(accs, accs, accs, accs, bias, Wn, alpha)


def _final_body(a_ref, b_ref, c_ref, d_ref, bias_ref, batch_ref,
                wl_ref, bl_ref, one_ref, out_ref, seg_ref, cnt_ref):
    i = pl.program_id(0)

    @pl.when(i == 0)
    def _():
        seg_ref[...] = jnp.zeros_like(seg_ref)
        cnt_ref[...] = jnp.zeros_like(cnt_ref)

    x4 = _pre((a_ref, b_ref, c_ref, d_ref), bias_ref, one_ref)
    brow = batch_ref[0]
    onehot = (lax.broadcasted_iota(jnp.int32, (G, BLK), 0) == brow
              ).astype(jnp.float32)
    seg_ref[...] += jnp.dot(onehot, x4, preferred_element_type=jnp.float32)
    cnt_ref[...] += jnp.sum(onehot, axis=1, keepdims=True)

    @pl.when(i == pl.num_programs(0) - 1)
    def _():
        pooled = seg_ref[...] / jnp.maximum(cnt_ref[...], 1.0)
        out_ref[...] = (
            jnp.dot(pooled, wl_ref[...], preferred_element_type=jnp.float32)
            + bl_ref[...])


def _final_call(accs, bias, batch3, Wl, bl):
    one = jnp.ones((1, 1), jnp.float32)
    return pl.pallas_call(
        _final_body,
        grid=(N // BLK,),
        in_specs=[
            pl.BlockSpec((1, BLK, QW), lambda i: (0, i, 0)),
            pl.BlockSpec((1, BLK, QW), lambda i: (1, i, 0)),
            pl.BlockSpec((1, BLK, QW), lambda i: (2, i, 0)),
            pl.BlockSpec((1, BLK, QW), lambda i: (3, i, 0)),
            pl.BlockSpec((1, H), lambda i: (0, 0)),
            pl.BlockSpec((1, 1, BLK), lambda i: (i, 0, 0)),
            pl.BlockSpec((H, C), lambda i: (0, 0)),
            pl.BlockSpec((1, C), lambda i: (0, 0)),
            pl.BlockSpec((1, 1), lambda i: (0, 0)),
        ],
        out_specs=pl.BlockSpec((G, C), lambda i: (0, 0)),
        out_shape=jax.ShapeDtypeStruct((G, C), jnp.float32),
        scratch_shapes=[
            pltpu.VMEM((G, H), jnp.float32),
            pltpu.VMEM((G, 1), jnp.float32),
        ],
        compiler_params=pltpu.CompilerParams(
            dimension_semantics=("arbitrary",)),
    )(accs, accs, accs, accs, bias, batch3, Wl, bl, one)


@jax.jit
def kernel(x, edge_index, edge_type, edge_attr, batch,
           W1, b1, W2, b2, W3, b3, W4, b4, Wl, bl):
    src = edge_index[0].reshape(NS, MCHUNKS, MK)
    dst = edge_index[1].reshape(NS, MCHUNKS, MK)
    srcd = edge_index[0].reshape(NS, CHUNKS, K)
    dstd = edge_index[1].reshape(NS, CHUNKS, K)
    wf = edge_attr[:, 0].reshape(NS, CHUNKS * K)
    srcn = edge_index[0].reshape(NC * NS, RPW, K)
    dstn = edge_index[1].reshape(NC * NS, RPW, K)
    wn = edge_attr[:, 0].reshape(NC * NS, RPW, K)

    deg_pad = _deg_kernel(srcd, dstd, wf).reshape(NC, N, L)
    norm = _norm_kernel(srcn, dstn, wn, deg_pad[0, :, 0],
                        deg_pad[1, :, 0]).reshape(NS, CHUNKS * K)

    accs0 = jnp.stack([x[:, q * QW:(q + 1) * QW] for q in range(4)])
    Ws = jnp.stack([W1, W2, W3, W4])
    bs = jnp.stack([jnp.zeros_like(b1), b1, b2, b3]).reshape(4, 1, H)
    alphas = jnp.array([0.0, 1.0, 1.0, 1.0], jnp.float32).reshape(4, 1, 1)

    def _layer_step(accs, wba):
        Wn, bprev, alpha = wba
        hq = _mid_call(accs, bprev, Wn, alpha)
        out = _msg_kernel(hq[0], hq[1], hq[2], hq[3], src, dst, norm)
        return out.reshape(4, N, QW), None

    accs, _ = lax.scan(_layer_step, accs0, (Ws, bs, alphas))

    batch3 = batch.reshape(N // BLK, 1, BLK)
    return _final_call(accs, b4.reshape(1, H), batch3, Wl,
                       bl.reshape(1, C))


# speedup vs baseline: 1.0205x; 1.0002x over previous
import functools

import jax
import jax.numpy as jnp
from jax import lax
from jax.experimental import pallas as pl
from jax.experimental.pallas import tpu as pltpu
from jax.experimental.pallas import tpu_sc as plsc

N = 10000
E = 320000
D = 128
H = 128
C = 10
G = 64

NC = 2
NS = 16
L = 16
QW = 32

K = 80
EPT = E // NS
CHUNKS = EPT // K
ROWS_E = E // K
NPT = N // NS
RPW = ROWS_E // (NC * NS)
MK = 125
MCHUNKS = EPT // MK

_MESH = plsc.VectorSubcoreMesh(core_axis_name="c", subcore_axis_name="s",
                               num_cores=NC, num_subcores=NS)
_SC_PARAMS = pltpu.CompilerParams(needs_layout_passes=False,
                                  use_tc_tiling_on_sc=False,
                                  disable_bounds_checks=True,
                                  disable_semaphore_checks=True)


@functools.partial(
    pl.kernel,
    out_type=jax.ShapeDtypeStruct((NC, NS, NPT, L), jnp.float32),
    mesh=_MESH,
    compiler_params=_SC_PARAMS,
    scratch_types=[
        pltpu.VMEM((CHUNKS, K), jnp.int32),
        pltpu.VMEM((CHUNKS * K,), jnp.float32),
        pltpu.VMEM((K, L), jnp.float32),
        pltpu.VMEM((K, L), jnp.float32),
        pltpu.VMEM((NPT, L), jnp.float32),
        pltpu.SemaphoreType.DMA,
        pltpu.SemaphoreType.DMA,
        pltpu.VMEM_SHARED((N, L), jnp.float32),
    ],
)
def _deg_kernel(src_hbm, dst_hbm, wf_hbm, deg_hbm, idx_v, w_v, pad0, pad1,
                zero_v, ps0, ps1, acc):
    c = lax.axis_index("c")
    s = lax.axis_index("s")

    @pl.when(c == 0)
    def _():
        pltpu.sync_copy(src_hbm.at[s], idx_v)

    @pl.when(c == 1)
    def _():
        pltpu.sync_copy(dst_hbm.at[s], idx_v)

    pltpu.sync_copy(wf_hbm.at[s], w_v)

    z16 = jnp.zeros((L,), jnp.float32)

    def zrow(i, _):
        zero_v[i, :] = z16
        return 0

    lax.fori_loop(0, NPT, zrow, 0)
    pltpu.sync_copy(zero_v, acc.at[pl.ds(s * NPT, NPT)])
    plsc.subcore_barrier()

    pads = ((pad0, ps0), (pad1, ps1))

    def pair(g2, _):
        for b in range(2):
            pv, sm = pads[b]
            g = g2 * 2 + b
            gK = g * K

            @pl.when(g >= 2)
            def _():
                pltpu.make_async_copy(pv, acc.at[idx_v.at[g - 2]], sm).wait()

            def row(r, _):
                pv[r, :] = plsc.load_gather(
                    w_v, [jnp.full((L,), gK + r, jnp.int32)])
                return 0

            lax.fori_loop(0, K, row, 0, unroll=8)
            pltpu.async_copy(pv, acc.at[idx_v.at[g]], sm, add=True)
        return 0

    lax.fori_loop(0, CHUNKS // 2, pair, 0)
    for g in (CHUNKS - 2, CHUNKS - 1):
        pv, sm = pads[g % 2]
        pltpu.make_async_copy(pv, acc.at[idx_v.at[g]], sm).wait()

    plsc.subcore_barrier()
    pltpu.sync_copy(acc.at[pl.ds(s * NPT, NPT)], deg_hbm.at[c, s])


def _rsqrt_body(ds_ref, dd_ref, rsd_ref, rdd_ref):
    rsd_ref[...] = lax.rsqrt(jnp.maximum(ds_ref[0][:, :1], 1e-12))
    rdd_ref[...] = lax.rsqrt(jnp.maximum(dd_ref[0][:, :1], 1e-12))


def _rsqrt_call(deg_pad):
    blk = 1000
    return pl.pallas_call(
        _rsqrt_body,
        grid=(N // blk,),
        in_specs=[
            pl.BlockSpec((1, blk, L), lambda i: (0, i, 0)),
            pl.BlockSpec((1, blk, L), lambda i: (1, i, 0)),
        ],
        out_specs=[
            pl.BlockSpec((blk, 1), lambda i: (i, 0)),
            pl.BlockSpec((blk, 1), lambda i: (i, 0)),
        ],
        out_shape=[
            jax.ShapeDtypeStruct((N, 1), jnp.float32),
            jax.ShapeDtypeStruct((N, 1), jnp.float32),
        ],
    )(deg_pad, deg_pad)


def _rsqrt16(x):
    i = plsc.bitcast(x, jnp.int32)
    i = jnp.int32(0x5F3759DF) - lax.shift_right_logical(i, 1)
    y = plsc.bitcast(i, jnp.float32)
    xh = x * 0.5
    for _ in range(3):
        y = y * (1.5 - xh * y * y)
    return y


@functools.partial(
    pl.kernel,
    out_type=jax.ShapeDtypeStruct((NC * NS, RPW, K), jnp.float32),
    mesh=_MESH,
    compiler_params=_SC_PARAMS,
    scratch_types=[
        pltpu.VMEM((N,), jnp.float32),
        pltpu.VMEM((N,), jnp.float32),
        pltpu.VMEM((RPW, K), jnp.int32),
        pltpu.VMEM((RPW, K), jnp.int32),
        pltpu.VMEM((RPW, K), jnp.float32),
        pltpu.VMEM((RPW, K), jnp.float32),
    ],
)
def _norm_kernel(src_hbm, dst_hbm, w_hbm, ds_hbm, dd_hbm, norm_hbm,
                 rsd_v, rdd_v, src_v, dst_v, w_v, norm_v):
    c = lax.axis_index("c")
    s = lax.axis_index("s")
    wid = s * NC + c

    pltpu.sync_copy(ds_hbm, rsd_v)
    pltpu.sync_copy(dd_hbm, rdd_v)
    pltpu.sync_copy(src_hbm.at[wid], src_v)
    pltpu.sync_copy(dst_hbm.at[wid], dst_v)
    pltpu.sync_copy(w_hbm.at[wid], w_v)

    def rrow(i, _):
        sl = pl.ds(i * L, L)
        rsd_v[sl] = _rsqrt16(jnp.maximum(rsd_v[sl], 1e-12))
        rdd_v[sl] = _rsqrt16(jnp.maximum(rdd_v[sl], 1e-12))
        return 0

    lax.fori_loop(0, N // L, rrow, 0, unroll=4)

    def row(g, _):
        for j in range(K // L):
            sl = pl.ds(j * L, L)
            a = plsc.load_gather(rsd_v, [src_v[g, sl]])
            b = plsc.load_gather(rdd_v, [dst_v[g, sl]])
            norm_v[g, sl] = w_v[g, sl] * a * b
        return 0

    lax.fori_loop(0, RPW, row, 0, unroll=4)
    pltpu.sync_copy(norm_v, norm_hbm.at[wid])


@functools.partial(
    pl.kernel,
    out_type=jax.ShapeDtypeStruct((2, NC, NS, NPT, QW), jnp.float32),
    mesh=_MESH,
    compiler_params=_SC_PARAMS,
    scratch_types=[
        pltpu.VMEM((MCHUNKS, MK), jnp.int32),
        pltpu.VMEM((MCHUNKS, MK), jnp.int32),
        pltpu.VMEM((MCHUNKS * MK,), jnp.float32),
        pltpu.VMEM((MK, QW), jnp.float32),
        pltpu.VMEM((MK, QW), jnp.float32),
        pltpu.VMEM((MK, QW), jnp.float32),
        pltpu.VMEM((MK, QW), jnp.float32),
        pltpu.VMEM((NPT, QW), jnp.float32),
        pltpu.SemaphoreType.DMA,
        pltpu.SemaphoreType.DMA,
        pltpu.SemaphoreType.DMA,
        pltpu.SemaphoreType.DMA,
        pltpu.SemaphoreType.DMA,
        pltpu.SemaphoreType.DMA,
        pltpu.SemaphoreType.DMA,
        pltpu.SemaphoreType.DMA,
        pltpu.VMEM_SHARED((N, QW), jnp.float32),
    ],
)
def _msg_kernel(h0_hbm, h1_hbm, h2_hbm, h3_hbm, src_hbm, dst_hbm,
                normf_hbm, out_hbm,
                src_v, dst_v, norm_v, rows0, rows1, rows2, rows3, zero_v,
                gs0, gs1, gs2, gs3, ss0, ss1, ss2, ss3, acc):
    c = lax.axis_index("c")
    s = lax.axis_index("s")

    pltpu.sync_copy(src_hbm.at[s], src_v)
    pltpu.sync_copy(dst_hbm.at[s], dst_v)
    pltpu.sync_copy(normf_hbm.at[s], norm_v)

    z16 = jnp.zeros((L,), jnp.float32)

    def zrow(i, _):
        for j in range(QW // L):
            zero_v[i, pl.ds(j * L, L)] = z16
        return 0

    lax.fori_loop(0, NPT, zrow, 0, unroll=8)

    bufs = ((rows0, gs0, ss0), (rows1, gs1, ss1), (rows2, gs2, ss2),
            (rows3, gs3, ss3))

    def body(h2_ref):
        for b in range(2):
            rv, gsm, _ = bufs[b]
            pltpu.async_copy(h2_ref.at[src_v.at[b]], rv, gsm)

        def quad(q, _):
            g0 = q * 4
            for b in range(4):
                rv, gsm, ssm = bufs[b]
                g = g0 + b
                pltpu.make_async_copy(h2_ref.at[src_v.at[g]], rv, gsm).wait()
                gK = g * MK

                def row(r, _):
                    nrm = plsc.load_gather(
                        norm_v, [jnp.full((L,), gK + r, jnp.int32)])
                    for j in range(QW // L):
                        sl = pl.ds(j * L, L)
                        rv[r, sl] = rv[r, sl] * nrm
                    return 0

                lax.fori_loop(0, MK, row, 0, unroll=8)
                pltpu.async_copy(rv, acc.at[dst_v.at[g]], ssm, add=True)

                rv2, gsm2, ssm2 = bufs[(b + 2) % 4]

                @pl.when(g >= 2)
                def _():
                    pltpu.make_async_copy(
                        rv2, acc.at[dst_v.at[g - 2]], ssm2).wait()

                @pl.when(g + 2 < MCHUNKS)
                def _():
                    pltpu.async_copy(h2_ref.at[src_v.at[g + 2]], rv2, gsm2)
            return 0

        lax.fori_loop(0, MCHUNKS // 4, quad, 0)

        for g in (MCHUNKS - 2, MCHUNKS - 1):
            rv, _, ssm = bufs[g % 4]
            pltpu.make_async_copy(rv, acc.at[dst_v.at[g]], ssm).wait()

    for p, (ha, hb) in enumerate(((h0_hbm, h1_hbm), (h2_hbm, h3_hbm))):
        pltpu.sync_copy(zero_v, acc.at[pl.ds(s * NPT, NPT)])
        plsc.subcore_barrier()

        @pl.when(c == 0)
        def _():
            body(ha)

        @pl.when(c == 1)
        def _():
            body(hb)

        plsc.subcore_barrier()
        pltpu.sync_copy(acc.at[pl.ds(s * NPT, NPT)], out_hbm.at[p, c, s])


BLK = 1000


def _elu(v):
    return jnp.where(v > 0, v, jnp.exp(jnp.minimum(v, 0.0)) - 1.0)


def _pre(q_refs, bias_ref, alpha_ref):
    xin = jnp.concatenate([q[0] for q in q_refs], axis=1)
    v = xin + bias_ref[...]
    return jnp.where(alpha_ref[...] > 0.0, _elu(v), v)


def _mid_body(a_ref, b_ref, c_ref, d_ref, bias_ref, w_ref, alpha_ref,
              h_ref):
    xin = _pre((a_ref, b_ref, c_ref, d_ref), bias_ref, alpha_ref)
    h = jnp.dot(xin, w_ref[...], preferred_element_type=jnp.float32)
    for q in range(4):
        h_ref[q] = h[:, q * QW:(q + 1) * QW]


def _mid_call(accs, bias, Wn, alpha):
    return pl.pallas_call(
        _mid_body,
        grid=(N // BLK,),
        in_specs=[
            pl.BlockSpec((1, BLK, QW), lambda i: (0, i, 0)),
            pl.BlockSpec((1, BLK, QW), lambda i: (1, i, 0)),
            pl.BlockSpec((1, BLK, QW), lambda i: (2, i, 0)),
            pl.BlockSpec((1, BLK, QW), lambda i: (3, i, 0)),
            pl.BlockSpec((1, H), lambda i: (0, 0)),
            pl.BlockSpec((H, H), lambda i: (0, 0)),
            pl.BlockSpec((1, 1), lambda i: (0, 0)),
        ],
        out_specs=pl.BlockSpec((4, BLK, QW), lambda i: (0, i, 0)),
        out_shape=jax.ShapeDtypeStruct((4, N, QW), jnp.float32),
    )(accs, accs, accs, accs, bias, Wn, alpha)


def _final_body(a_ref, b_ref, c_ref, d_ref, bias_ref, batch_ref,
                wl_ref, bl_ref, one_ref, out_ref, seg_ref, cnt_ref):
    i = pl.program_id(0)

    @pl.when(i == 0)
    def _():
        seg_ref[...] = jnp.zeros_like(seg_ref)
        cnt_ref[...] = jnp.zeros_like(cnt_ref)

    x4 = _pre((a_ref, b_ref, c_ref, d_ref), bias_ref, one_ref)
    brow = batch_ref[0]
    onehot = (lax.broadcasted_iota(jnp.int32, (G, BLK), 0) == brow
              ).astype(jnp.float32)
    seg_ref[...] += jnp.dot(onehot, x4, preferred_element_type=jnp.float32)
    cnt_ref[...] += jnp.sum(onehot, axis=1, keepdims=True)

    @pl.when(i == pl.num_programs(0) - 1)
    def _():
        pooled = seg_ref[...] / jnp.maximum(cnt_ref[...], 1.0)
        out_ref[...] = (
            jnp.dot(pooled, wl_ref[...], preferred_element_type=jnp.float32)
            + bl_ref[...])


def _final_call(accs, bias, batch3, Wl, bl):
    one = jnp.ones((1, 1), jnp.float32)
    return pl.pallas_call(
        _final_body,
        grid=(N // BLK,),
        in_specs=[
            pl.BlockSpec((1, BLK, QW), lambda i: (0, i, 0)),
            pl.BlockSpec((1, BLK, QW), lambda i: (1, i, 0)),
            pl.BlockSpec((1, BLK, QW), lambda i: (2, i, 0)),
            pl.BlockSpec((1, BLK, QW), lambda i: (3, i, 0)),
            pl.BlockSpec((1, H), lambda i: (0, 0)),
            pl.BlockSpec((1, 1, BLK), lambda i: (i, 0, 0)),
            pl.BlockSpec((H, C), lambda i: (0, 0)),
            pl.BlockSpec((1, C), lambda i: (0, 0)),
            pl.BlockSpec((1, 1), lambda i: (0, 0)),
        ],
        out_specs=pl.BlockSpec((G, C), lambda i: (0, 0)),
        out_shape=jax.ShapeDtypeStruct((G, C), jnp.float32),
        scratch_shapes=[
            pltpu.VMEM((G, H), jnp.float32),
            pltpu.VMEM((G, 1), jnp.float32),
        ],
        compiler_params=pltpu.CompilerParams(
            dimension_semantics=("arbitrary",)),
    )(accs, accs, accs, accs, bias, batch3, Wl, bl, one)


@jax.jit
def kernel(x, edge_index, edge_type, edge_attr, batch,
           W1, b1, W2, b2, W3, b3, W4, b4, Wl, bl):
    src = edge_index[0].reshape(NS, MCHUNKS, MK)
    dst = edge_index[1].reshape(NS, MCHUNKS, MK)
    srcd = edge_index[0].reshape(NS, CHUNKS, K)
    dstd = edge_index[1].reshape(NS, CHUNKS, K)
    wf = edge_attr[:, 0].reshape(NS, CHUNKS * K)
    srcn = edge_index[0].reshape(NC * NS, RPW, K)
    dstn = edge_index[1].reshape(NC * NS, RPW, K)
    wn = edge_attr[:, 0].reshape(NC * NS, RPW, K)

    deg_pad = _deg_kernel(srcd, dstd, wf).reshape(NC, N, L)
    norm = _norm_kernel(srcn, dstn, wn, deg_pad[0, :, 0],
                        deg_pad[1, :, 0]).reshape(NS, CHUNKS * K)

    accs0 = jnp.stack([x[:, q * QW:(q + 1) * QW] for q in range(4)])
    Ws = jnp.stack([W1, W2, W3, W4])
    bs = jnp.stack([jnp.zeros_like(b1), b1, b2, b3]).reshape(4, 1, H)
    alphas = jnp.array([0.0, 1.0, 1.0, 1.0], jnp.float32).reshape(4, 1, 1)

    def _layer_step(accs, wba):
        Wn, bprev, alpha = wba
        hq = _mid_call(accs, bprev, Wn, alpha)
        out = _msg_kernel(hq[0], hq[1], hq[2], hq[3], src, dst, norm)
        return out.reshape(4, N, QW), None

    accs, _ = lax.scan(_layer_step, accs0, (Ws, bs, alphas))

    batch3 = batch.reshape(N // BLK, 1, BLK)
    return _final_call(accs, b4.reshape(1, H), batch3, Wl,
                       bl.reshape(1, C))
